# baseline (speedup 1.0000x reference)
import jax
import jax.numpy as jnp
from jax import lax
from jax.experimental import pallas as pl

_PROMPT_LENGTH = 20
_EMBED_DIM = 64
_SEQ_LEN = 220
_BATCH_GROUP = 16
_PACK = 4
_K = 2 * _PROMPT_LENGTH


def _body(idx_ref, tbl_ref, out_ref):
    n = idx_ref.shape[-1]
    parts = []
    for i in range(_PACK):
        row = idx_ref[0, i:i + 1, :]
        parts.append(row == lax.broadcasted_iota(jnp.int32, (_K, n), 0))
    onehot_t = jnp.concatenate(parts, axis=0).astype(jnp.float32)
    out_ref[...] = lax.dot_general(
        onehot_t, tbl_ref[...], (((0,), (0,)), ((), ())),
        preferred_element_type=jnp.float32)


def kernel(input, shared_weight, prompt_weight):
    batch, seq_len = input.shape
    total = batch * seq_len
    n = (_BATCH_GROUP * seq_len) // _PACK
    n_groups = batch // _BATCH_GROUP

    off = jnp.where(jnp.arange(seq_len) >= _PROMPT_LENGTH,
                    _PROMPT_LENGTH, 0).astype(jnp.int32)
    adj = input.astype(jnp.int32) + off[None, :]
    idx4 = jnp.moveaxis(adj.reshape(n_groups * n, _PACK), -1, 0)
    idx4 = idx4.reshape(_PACK, n_groups, n).swapaxes(0, 1)

    tbl = jnp.concatenate(
        [prompt_weight, shared_weight[:_PROMPT_LENGTH]], axis=0)
    btbl = jnp.zeros((_PACK * _K, _PACK * _EMBED_DIM), jnp.float32)
    for i in range(_PACK):
        btbl = btbl.at[i * _K:(i + 1) * _K,
                       i * _EMBED_DIM:(i + 1) * _EMBED_DIM].set(tbl)

    out = pl.pallas_call(
        _body,
        grid=(n_groups,),
        in_specs=[
            pl.BlockSpec((1, _PACK, n), lambda i: (i, 0, 0)),
            pl.BlockSpec((_PACK * _K, _PACK * _EMBED_DIM), lambda i: (0, 0)),
        ],
        out_specs=pl.BlockSpec((n, _PACK * _EMBED_DIM), lambda i: (i, 0)),
        out_shape=jax.ShapeDtypeStruct(
            (total // _PACK, _PACK * _EMBED_DIM), jnp.float32),
    )(idx4, btbl)
    return out.reshape(batch, seq_len, _EMBED_DIM)
